# baseline (speedup 1.0000x reference)
import jax
import jax.numpy as jnp
from jax import lax
from jax.experimental import pallas as pl
from jax.experimental.pallas import tpu as pltpu
from jax.experimental.pallas import tpu_sc as plsc

B = 16
D = 1024
V = 100000
L = 200
TOP_K = 50
MIN_KEEP = 5

VT = 7168
COL0 = 7 * VT
W1 = V - COL0
CAP = 2048
PHYS = CAP + 16
KEEP = 256
OUT_CAP = 512
HL = 256
NBIS = 20
NEG = -1e30
NEG2 = -3e30


def _matmul_body(h_ref, gamma_ref, beta_ref, w_ref, out_ref):
    x = h_ref[...]
    mu = jnp.mean(x, axis=-1, keepdims=True)
    var = jnp.mean((x - mu) ** 2, axis=-1, keepdims=True)
    hn = (x - mu) * lax.rsqrt(var + 1e-5) * gamma_ref[...] + beta_ref[...]
    out_ref[...] = lax.dot_general(
        hn, w_ref[...], (((1,), (1,)), ((), ())),
        preferred_element_type=jnp.float32,
    )


def _logits_chunk(hidden_states, ln_gamma, ln_beta, lm_w, col0, width):
    blk0 = col0 // VT
    return pl.pallas_call(
        _matmul_body,
        grid=(pl.cdiv(width, VT),),
        in_specs=[
            pl.BlockSpec((B, D), lambda i: (0, 0)),
            pl.BlockSpec((D,), lambda i: (0,)),
            pl.BlockSpec((D,), lambda i: (0,)),
            pl.BlockSpec((VT, D), lambda i: (i + blk0, 0)),
        ],
        out_specs=pl.BlockSpec((B, VT), lambda i: (0, i)),
        out_shape=jax.ShapeDtypeStruct((B, width), jnp.float32),
    )(hidden_states, ln_gamma, ln_beta, lm_w)


def _sortable(s):
    return s ^ (lax.shift_right_arithmetic(s, 31) & jnp.int32(0x7FFFFFFF))


def _make_sc_body(col0, width, part0):
    segc = width // 2
    nv = segc // 16

    def body(logits_ref, ids_ref, ov_ref, oi_ref, oh_ref,
             seg_v, bval_v, bidx_v, sort_v, ids_v, hist_v, dma_sem):
        row = lax.axis_index("s")
        half = lax.axis_index("c")
        gbase = col0 + half * segc
        iota = lax.iota(jnp.int32, 16)

        pltpu.sync_copy(logits_ref.at[pl.ds(row * width + half * segc, segc)],
                        seg_v)

        pltpu.sync_copy(ids_ref.at[pl.ds(row * HL, HL)], ids_v)
        for k in range(HL // 16):
            idv = ids_v[pl.ds(k * 16, 16)]
            safe = jnp.clip(idv - gbase, 0, segc - 1)
            hist_v[pl.ds(k * 16, 16)] = plsc.load_gather(seg_v, [safe])
        pltpu.sync_copy(hist_v, oh_ref.at[pl.ds((row * 2 + half) * HL, HL)])

        def count_gt(thr_s):
            def cbody(k, acc):
                s = sort_v[pl.ds(k * 16, 16)]
                return acc + jnp.where(s > thr_s, jnp.int32(1), jnp.int32(0))
            acc = lax.fori_loop(0, PHYS // 16, cbody,
                                jnp.zeros((16,), jnp.int32))
            return jnp.sum(acc)

        def tighten(cnt, thr):
            def conv(k, _):
                w = bval_v[pl.ds(k * 16, 16)]
                s = _sortable(plsc.bitcast(w, jnp.int32))
                pos = k * 16 + iota
                sort_v[pl.ds(k * 16, 16)] = jnp.where(pos < cnt, s,
                                                      jnp.int32(-2**31))
                return 0
            lax.fori_loop(0, PHYS // 16, conv, 0)

            def bis(_, lohi):
                lo, hi = lohi
                mid = lo + lax.shift_right_logical(hi - lo, 1)
                pred = count_gt(mid) >= KEEP
                return (jnp.where(pred, mid, lo), jnp.where(pred, hi, mid))
            lo, hi = lax.fori_loop(
                0, NBIS, bis,
                (jnp.int32(-2139095041), jnp.int32(2139095040)))

            def comp(k, newcnt):
                s = sort_v[pl.ds(k * 16, 16)]
                m2 = s > lo
                w = bval_v[pl.ds(k * 16, 16)]
                x = bidx_v[pl.ds(k * 16, 16)]
                plsc.store_compressed(bval_v.at[pl.ds(newcnt, 16)], w,
                                      mask=m2)
                plsc.store_compressed(bidx_v.at[pl.ds(newcnt, 16)], x,
                                      mask=m2)
                return newcnt + jnp.max(plsc.all_reduce_population_count(m2))
            newcnt = lax.fori_loop(0, PHYS // 16, comp, jnp.int32(0))
            thr_f = plsc.bitcast(_sortable(jnp.full((16,), lo, jnp.int32)),
                                 jnp.float32)
            return newcnt, thr_f

        def append_one(i, cnt, thr):
            v = seg_v[pl.ds(i * 16, 16)]
            m = v > thr
            n = jnp.max(plsc.all_reduce_population_count(m))
            plsc.store_compressed(bval_v.at[pl.ds(cnt, 16)], v, mask=m)
            plsc.store_compressed(bidx_v.at[pl.ds(cnt, 16)],
                                  gbase + i * 16 + iota, mask=m)
            return cnt + n

        def fillk(k, _):
            bval_v[pl.ds(k * 16, 16)] = seg_v[pl.ds(k * 16, 16)]
            bidx_v[pl.ds(k * 16, 16)] = gbase + k * 16 + iota
            return 0
        lax.fori_loop(0, CAP // 16, fillk, 0)
        thr0 = jnp.full((16,), NEG, jnp.float32)
        cnt, thr = tighten(jnp.int32(CAP), thr0)

        nblk = (nv - CAP // 16) // 4
        tail0 = CAP // 16 + nblk * 4

        def blk(b, carry):
            cnt, thr = carry
            i0 = CAP // 16 + b * 4
            vs, ms = [], []
            anym = None
            for u in range(4):
                v = seg_v[pl.ds((i0 + u) * 16, 16)]
                m = v > thr
                vs.append(v)
                ms.append(m)
                anym = m if anym is None else (anym | m)

            def hit(c):
                for u in range(4):
                    n = jnp.max(plsc.all_reduce_population_count(ms[u]))
                    plsc.store_compressed(bval_v.at[pl.ds(c, 16)], vs[u],
                                          mask=ms[u])
                    plsc.store_compressed(bidx_v.at[pl.ds(c, 16)],
                                          gbase + (i0 + u) * 16 + iota,
                                          mask=ms[u])
                    c = c + n
                return c
            cnt = lax.cond(jnp.any(anym), hit, lambda c: c, cnt)
            return lax.cond(cnt >= CAP - 64, tighten, lambda c, t: (c, t),
                            cnt, thr)
        cnt, thr = lax.fori_loop(0, nblk, blk, (cnt, thr))
        for i in range(tail0, nv):
            cnt = append_one(i, cnt, thr)

        cnt, thr = lax.cond(cnt > OUT_CAP, tighten, lambda c, t: (c, t),
                            cnt, thr)

        def padk(k, _):
            pos = k * 16 + iota
            valid = pos < cnt
            w = bval_v[pl.ds(k * 16, 16)]
            x = bidx_v[pl.ds(k * 16, 16)]
            bval_v[pl.ds(k * 16, 16)] = jnp.where(valid, w, NEG)
            bidx_v[pl.ds(k * 16, 16)] = jnp.where(
                valid, x, V + (part0 + half) * OUT_CAP + pos)
            return 0
        lax.fori_loop(0, OUT_CAP // 16, padk, 0)
        obase = (row * 2 + half) * OUT_CAP
        pltpu.sync_copy(bval_v.at[pl.ds(0, OUT_CAP)],
                        ov_ref.at[pl.ds(obase, OUT_CAP)])
        pltpu.sync_copy(bidx_v.at[pl.ds(0, OUT_CAP)],
                        oi_ref.at[pl.ds(obase, OUT_CAP)])

    return body


def _sc_collect(logits_chunk, ids_pad, col0, width, part0):
    mesh = plsc.VectorSubcoreMesh(
        core_axis_name="c", subcore_axis_name="s",
        num_cores=2, num_subcores=16)
    f = pl.kernel(
        _make_sc_body(col0, width, part0),
        out_type=[
            jax.ShapeDtypeStruct((B * 2 * OUT_CAP,), jnp.float32),
            jax.ShapeDtypeStruct((B * 2 * OUT_CAP,), jnp.int32),
            jax.ShapeDtypeStruct((B * 2 * HL,), jnp.float32),
        ],
        mesh=mesh,
        scratch_types=[
            pltpu.VMEM((width // 2,), jnp.float32),
            pltpu.VMEM((PHYS,), jnp.float32),
            pltpu.VMEM((PHYS,), jnp.int32),
            pltpu.VMEM((PHYS,), jnp.int32),
            pltpu.VMEM((HL,), jnp.int32),
            pltpu.VMEM((HL,), jnp.float32),
            pltpu.SemaphoreType.DMA,
        ],
        compiler_params=pltpu.CompilerParams(
            use_tc_tiling_on_sc=False, needs_layout_passes=False),
    )
    return f(logits_chunk.reshape(B * width), ids_pad.reshape(B * HL))


NCAND = 4 * OUT_CAP + HL
S0 = COL0 // 2
S1 = COL0
S2 = COL0 + W1 // 2


def _merge_body(cv0_ref, ci0_ref, cv1_ref, ci1_ref, oh0_ref, oh1_ref,
                ids_ref, topp_ref, temp_ref, pen_ref,
                probs_ref, token_ref):
    ids = ids_ref[...]
    jj = lax.broadcasted_iota(jnp.int32, (B, HL), 1)
    valid = jj < L

    oh0 = oh0_ref[...]
    oh1 = oh1_ref[...]
    hv = jnp.where(
        ids < S1,
        jnp.where(ids < S0, oh0[:, :HL], oh0[:, HL:]),
        jnp.where(ids < S2, oh1[:, :HL], oh1[:, HL:]))
    pen = pen_ref[0, 0]
    hv = jnp.where(hv < 0, hv * pen, hv / pen)

    dup = ~valid
    for j in range(L):
        pj = ids[:, j:j + 1]
        dup = dup | ((ids == pj) & (jj > j))
    hv = jnp.where(dup, NEG, hv)
    hidx = jnp.where(dup, V + 4 * OUT_CAP + jj, ids)

    cv = jnp.concatenate([cv0_ref[...], cv1_ref[...]], axis=1)
    ci = jnp.concatenate([ci0_ref[...], ci1_ref[...]], axis=1)
    ex = jnp.zeros(cv.shape, jnp.bool_)
    for j in range(L):
        ex = ex | (ci == ids[:, j:j + 1])
    cv = jnp.where(ex, NEG, cv)

    allv = jnp.concatenate([cv, hv], axis=1)
    alli = jnp.concatenate([ci, hidx], axis=1)

    lane = lax.broadcasted_iota(jnp.int32, (B, 64), 1)
    def xbody(t, carry):
        ovals, otok, av = carry
        m = jnp.max(av, axis=1, keepdims=True)
        elig = av == m
        sel = jnp.min(jnp.where(elig, alli, jnp.int32(2**31 - 1)),
                      axis=1, keepdims=True)
        av = jnp.where(elig & (alli == sel), NEG2, av)
        ovals = jnp.where(lane == t, m, ovals)
        otok = jnp.where(lane == t, sel, otok)
        return ovals, otok, av
    ovals0 = jnp.full((B, 64), NEG, jnp.float32)
    otok0 = jnp.zeros((B, 64), jnp.int32)
    ovals, otok, _ = lax.fori_loop(0, TOP_K, xbody, (ovals0, otok0, allv))

    act = lane < TOP_K
    tl = ovals / temp_ref[0, 0]
    m1 = jnp.max(jnp.where(act, tl, NEG), axis=1, keepdims=True)
    e = jnp.where(act, jnp.exp(tl - m1), 0.0)
    p = e / jnp.sum(e, axis=1, keepdims=True)
    r64 = lax.broadcasted_iota(jnp.int32, (64, 64), 0)
    c64 = lax.broadcasted_iota(jnp.int32, (64, 64), 1)
    tri = (r64 <= c64).astype(jnp.float32)
    cum = lax.dot_general(p, tri, (((1,), (0,)), ((), ())),
                          preferred_element_type=jnp.float32,
                          precision=lax.Precision.HIGHEST)
    keep = (cum < topp_ref[0, 0]) | (lane < MIN_KEEP)
    fl = jnp.where(keep, tl, jnp.float32(-1000.0))
    m2 = jnp.max(jnp.where(act, fl, NEG), axis=1, keepdims=True)
    e2 = jnp.where(act, jnp.exp(fl - m2), 0.0)
    pf = e2 / jnp.sum(e2, axis=1, keepdims=True)

    probs_ref[...] = pf[:, :TOP_K]
    token_ref[...] = otok[:, :TOP_K]


def _merge(c0, c1, ids_pad, top_p, temperature, penalty):
    cv0, ci0, oh0 = c0
    cv1, ci1, oh1 = c1
    return pl.pallas_call(
        _merge_body,
        out_shape=(
            jax.ShapeDtypeStruct((B, TOP_K), jnp.float32),
            jax.ShapeDtypeStruct((B, TOP_K), jnp.int32),
        ),
    )(cv0.reshape(B, 2 * OUT_CAP), ci0.reshape(B, 2 * OUT_CAP),
      cv1.reshape(B, 2 * OUT_CAP), ci1.reshape(B, 2 * OUT_CAP),
      oh0.reshape(B, 2 * HL), oh1.reshape(B, 2 * HL),
      ids_pad, top_p.reshape(1, 1), temperature.reshape(1, 1),
      penalty.reshape(1, 1))


def kernel(hidden_states, input_ids, top_p, temperature, penalty,
           ln_gamma, ln_beta, lm_w):
    ids32 = input_ids.astype(jnp.int32)
    ids_pad = jnp.pad(ids32, ((0, 0), (0, HL - L)))
    lg0 = _logits_chunk(hidden_states, ln_gamma, ln_beta, lm_w, 0, COL0)
    c0 = _sc_collect(lg0, ids_pad, 0, COL0, 0)
    lg1 = _logits_chunk(hidden_states, ln_gamma, ln_beta, lm_w, COL0, W1)
    c1 = _sc_collect(lg1, ids_pad, COL0, W1, 2)
    probs, token = _merge(c0, c1, ids_pad, top_p, temperature, penalty)
    return (probs, token)


# speedup vs baseline: 1.0343x; 1.0343x over previous
import jax
import jax.numpy as jnp
from jax import lax
from jax.experimental import pallas as pl
from jax.experimental.pallas import tpu as pltpu
from jax.experimental.pallas import tpu_sc as plsc

B = 16
D = 1024
V = 100000
L = 200
TOP_K = 50
MIN_KEEP = 5

VT = 3584
COL0 = 14 * VT
W1 = V - COL0
CAP = 2048
PHYS = CAP + 16
KEEP = 256
OUT_CAP = 512
HL = 256
NBIS = 20
NEG = -1e30
NEG2 = -3e30


def _matmul_body(h_ref, gamma_ref, beta_ref, w_ref, out_ref):
    x = h_ref[...]
    mu = jnp.mean(x, axis=-1, keepdims=True)
    var = jnp.mean((x - mu) ** 2, axis=-1, keepdims=True)
    hn = (x - mu) * lax.rsqrt(var + 1e-5) * gamma_ref[...] + beta_ref[...]
    out_ref[...] = lax.dot_general(
        hn, w_ref[...], (((1,), (1,)), ((), ())),
        preferred_element_type=jnp.float32,
    )


def _logits_chunk(hidden_states, ln_gamma, ln_beta, lm_w, col0, width):
    blk0 = col0 // VT
    return pl.pallas_call(
        _matmul_body,
        grid=(pl.cdiv(width, VT),),
        in_specs=[
            pl.BlockSpec((B, D), lambda i: (0, 0)),
            pl.BlockSpec((D,), lambda i: (0,)),
            pl.BlockSpec((D,), lambda i: (0,)),
            pl.BlockSpec((VT, D), lambda i: (i + blk0, 0)),
        ],
        out_specs=pl.BlockSpec((B, VT), lambda i: (0, i)),
        out_shape=jax.ShapeDtypeStruct((B, width), jnp.float32),
    )(hidden_states, ln_gamma, ln_beta, lm_w)


def _sortable(s):
    return s ^ (lax.shift_right_arithmetic(s, 31) & jnp.int32(0x7FFFFFFF))


def _make_sc_body(col0, width, part0):
    segc = width // 2
    nv = segc // 16

    def body(logits_ref, ids_ref, ov_ref, oi_ref, oh_ref,
             seg_v, bval_v, bidx_v, sort_v, ids_v, hist_v, dma_sem):
        row = lax.axis_index("s")
        half = lax.axis_index("c")
        gbase = col0 + half * segc
        iota = lax.iota(jnp.int32, 16)

        pltpu.sync_copy(logits_ref.at[pl.ds(row * width + half * segc, segc)],
                        seg_v)

        pltpu.sync_copy(ids_ref.at[pl.ds(row * HL, HL)], ids_v)
        for k in range(HL // 16):
            idv = ids_v[pl.ds(k * 16, 16)]
            safe = jnp.clip(idv - gbase, 0, segc - 1)
            hist_v[pl.ds(k * 16, 16)] = plsc.load_gather(seg_v, [safe])
        pltpu.sync_copy(hist_v, oh_ref.at[pl.ds((row * 2 + half) * HL, HL)])

        def count_gt(thr_s):
            def cbody(k, acc):
                s = sort_v[pl.ds(k * 16, 16)]
                return acc + jnp.where(s > thr_s, jnp.int32(1), jnp.int32(0))
            acc = lax.fori_loop(0, PHYS // 16, cbody,
                                jnp.zeros((16,), jnp.int32))
            return jnp.sum(acc)

        def tighten(cnt, thr):
            def conv(k, _):
                w = bval_v[pl.ds(k * 16, 16)]
                s = _sortable(plsc.bitcast(w, jnp.int32))
                pos = k * 16 + iota
                sort_v[pl.ds(k * 16, 16)] = jnp.where(pos < cnt, s,
                                                      jnp.int32(-2**31))
                return 0
            lax.fori_loop(0, PHYS // 16, conv, 0)

            def bis(_, lohi):
                lo, hi = lohi
                mid = lo + lax.shift_right_logical(hi - lo, 1)
                pred = count_gt(mid) >= KEEP
                return (jnp.where(pred, mid, lo), jnp.where(pred, hi, mid))
            lo, hi = lax.fori_loop(
                0, NBIS, bis,
                (jnp.int32(-2139095041), jnp.int32(2139095040)))

            def comp(k, newcnt):
                s = sort_v[pl.ds(k * 16, 16)]
                m2 = s > lo
                w = bval_v[pl.ds(k * 16, 16)]
                x = bidx_v[pl.ds(k * 16, 16)]
                plsc.store_compressed(bval_v.at[pl.ds(newcnt, 16)], w,
                                      mask=m2)
                plsc.store_compressed(bidx_v.at[pl.ds(newcnt, 16)], x,
                                      mask=m2)
                return newcnt + jnp.max(plsc.all_reduce_population_count(m2))
            newcnt = lax.fori_loop(0, PHYS // 16, comp, jnp.int32(0))
            thr_f = plsc.bitcast(_sortable(jnp.full((16,), lo, jnp.int32)),
                                 jnp.float32)
            return newcnt, thr_f

        def append_one(i, cnt, thr):
            v = seg_v[pl.ds(i * 16, 16)]
            m = v > thr
            n = jnp.max(plsc.all_reduce_population_count(m))
            plsc.store_compressed(bval_v.at[pl.ds(cnt, 16)], v, mask=m)
            plsc.store_compressed(bidx_v.at[pl.ds(cnt, 16)],
                                  gbase + i * 16 + iota, mask=m)
            return cnt + n

        def fillk(k, _):
            bval_v[pl.ds(k * 16, 16)] = seg_v[pl.ds(k * 16, 16)]
            bidx_v[pl.ds(k * 16, 16)] = gbase + k * 16 + iota
            return 0
        lax.fori_loop(0, CAP // 16, fillk, 0)
        thr0 = jnp.full((16,), NEG, jnp.float32)
        cnt, thr = tighten(jnp.int32(CAP), thr0)

        nblk = (nv - CAP // 16) // 4
        tail0 = CAP // 16 + nblk * 4

        def blk(b, carry):
            cnt, thr = carry
            i0 = CAP // 16 + b * 4
            vs, ms = [], []
            anym = None
            for u in range(4):
                v = seg_v[pl.ds((i0 + u) * 16, 16)]
                m = v > thr
                vs.append(v)
                ms.append(m)
                anym = m if anym is None else (anym | m)

            def hit(c):
                for u in range(4):
                    n = jnp.max(plsc.all_reduce_population_count(ms[u]))
                    plsc.store_compressed(bval_v.at[pl.ds(c, 16)], vs[u],
                                          mask=ms[u])
                    plsc.store_compressed(bidx_v.at[pl.ds(c, 16)],
                                          gbase + (i0 + u) * 16 + iota,
                                          mask=ms[u])
                    c = c + n
                return c
            cnt = lax.cond(jnp.any(anym), hit, lambda c: c, cnt)
            return lax.cond(cnt >= CAP - 64, tighten, lambda c, t: (c, t),
                            cnt, thr)
        cnt, thr = lax.fori_loop(0, nblk, blk, (cnt, thr))
        for i in range(tail0, nv):
            cnt = append_one(i, cnt, thr)

        cnt, thr = lax.cond(cnt > OUT_CAP, tighten, lambda c, t: (c, t),
                            cnt, thr)

        def padk(k, _):
            pos = k * 16 + iota
            valid = pos < cnt
            w = bval_v[pl.ds(k * 16, 16)]
            x = bidx_v[pl.ds(k * 16, 16)]
            bval_v[pl.ds(k * 16, 16)] = jnp.where(valid, w, NEG)
            bidx_v[pl.ds(k * 16, 16)] = jnp.where(
                valid, x, V + (part0 + half) * OUT_CAP + pos)
            return 0
        lax.fori_loop(0, OUT_CAP // 16, padk, 0)
        obase = (row * 2 + half) * OUT_CAP
        pltpu.sync_copy(bval_v.at[pl.ds(0, OUT_CAP)],
                        ov_ref.at[pl.ds(obase, OUT_CAP)])
        pltpu.sync_copy(bidx_v.at[pl.ds(0, OUT_CAP)],
                        oi_ref.at[pl.ds(obase, OUT_CAP)])

    return body


def _sc_collect(logits_chunk, ids_pad, col0, width, part0):
    mesh = plsc.VectorSubcoreMesh(
        core_axis_name="c", subcore_axis_name="s",
        num_cores=2, num_subcores=16)
    f = pl.kernel(
        _make_sc_body(col0, width, part0),
        out_type=[
            jax.ShapeDtypeStruct((B * 2 * OUT_CAP,), jnp.float32),
            jax.ShapeDtypeStruct((B * 2 * OUT_CAP,), jnp.int32),
            jax.ShapeDtypeStruct((B * 2 * HL,), jnp.float32),
        ],
        mesh=mesh,
        scratch_types=[
            pltpu.VMEM((width // 2,), jnp.float32),
            pltpu.VMEM((PHYS,), jnp.float32),
            pltpu.VMEM((PHYS,), jnp.int32),
            pltpu.VMEM((PHYS,), jnp.int32),
            pltpu.VMEM((HL,), jnp.int32),
            pltpu.VMEM((HL,), jnp.float32),
            pltpu.SemaphoreType.DMA,
        ],
        compiler_params=pltpu.CompilerParams(
            use_tc_tiling_on_sc=False, needs_layout_passes=False),
    )
    return f(logits_chunk.reshape(B * width), ids_pad.reshape(B * HL))


NCAND = 4 * OUT_CAP + HL
S0 = COL0 // 2
S1 = COL0
S2 = COL0 + W1 // 2


def _merge_body(cv0_ref, ci0_ref, cv1_ref, ci1_ref, oh0_ref, oh1_ref,
                ids_ref, topp_ref, temp_ref, pen_ref,
                probs_ref, token_ref):
    ids = ids_ref[...]
    jj = lax.broadcasted_iota(jnp.int32, (B, HL), 1)
    valid = jj < L

    oh0 = oh0_ref[...]
    oh1 = oh1_ref[...]
    hv = jnp.where(
        ids < S1,
        jnp.where(ids < S0, oh0[:, :HL], oh0[:, HL:]),
        jnp.where(ids < S2, oh1[:, :HL], oh1[:, HL:]))
    pen = pen_ref[0, 0]
    hv = jnp.where(hv < 0, hv * pen, hv / pen)

    dup = ~valid
    for j in range(L):
        pj = ids[:, j:j + 1]
        dup = dup | ((ids == pj) & (jj > j))
    hv = jnp.where(dup, NEG, hv)
    hidx = jnp.where(dup, V + 4 * OUT_CAP + jj, ids)

    cv = jnp.concatenate([cv0_ref[...], cv1_ref[...]], axis=1)
    ci = jnp.concatenate([ci0_ref[...], ci1_ref[...]], axis=1)
    ex = jnp.zeros(cv.shape, jnp.bool_)
    for j in range(L):
        ex = ex | (ci == ids[:, j:j + 1])
    cv = jnp.where(ex, NEG, cv)

    allv = jnp.concatenate([cv, hv], axis=1)
    alli = jnp.concatenate([ci, hidx], axis=1)

    lane = lax.broadcasted_iota(jnp.int32, (B, 64), 1)
    def xbody(t, carry):
        ovals, otok, av = carry
        m = jnp.max(av, axis=1, keepdims=True)
        elig = av == m
        sel = jnp.min(jnp.where(elig, alli, jnp.int32(2**31 - 1)),
                      axis=1, keepdims=True)
        av = jnp.where(elig & (alli == sel), NEG2, av)
        ovals = jnp.where(lane == t, m, ovals)
        otok = jnp.where(lane == t, sel, otok)
        return ovals, otok, av
    ovals0 = jnp.full((B, 64), NEG, jnp.float32)
    otok0 = jnp.zeros((B, 64), jnp.int32)
    ovals, otok, _ = lax.fori_loop(0, TOP_K, xbody, (ovals0, otok0, allv))

    act = lane < TOP_K
    tl = ovals / temp_ref[0, 0]
    m1 = jnp.max(jnp.where(act, tl, NEG), axis=1, keepdims=True)
    e = jnp.where(act, jnp.exp(tl - m1), 0.0)
    p = e / jnp.sum(e, axis=1, keepdims=True)
    r64 = lax.broadcasted_iota(jnp.int32, (64, 64), 0)
    c64 = lax.broadcasted_iota(jnp.int32, (64, 64), 1)
    tri = (r64 <= c64).astype(jnp.float32)
    cum = lax.dot_general(p, tri, (((1,), (0,)), ((), ())),
                          preferred_element_type=jnp.float32,
                          precision=lax.Precision.HIGHEST)
    keep = (cum < topp_ref[0, 0]) | (lane < MIN_KEEP)
    fl = jnp.where(keep, tl, jnp.float32(-1000.0))
    m2 = jnp.max(jnp.where(act, fl, NEG), axis=1, keepdims=True)
    e2 = jnp.where(act, jnp.exp(fl - m2), 0.0)
    pf = e2 / jnp.sum(e2, axis=1, keepdims=True)

    probs_ref[...] = pf[:, :TOP_K]
    token_ref[...] = otok[:, :TOP_K]


def _merge(c0, c1, ids_pad, top_p, temperature, penalty):
    cv0, ci0, oh0 = c0
    cv1, ci1, oh1 = c1
    return pl.pallas_call(
        _merge_body,
        out_shape=(
            jax.ShapeDtypeStruct((B, TOP_K), jnp.float32),
            jax.ShapeDtypeStruct((B, TOP_K), jnp.int32),
        ),
    )(cv0.reshape(B, 2 * OUT_CAP), ci0.reshape(B, 2 * OUT_CAP),
      cv1.reshape(B, 2 * OUT_CAP), ci1.reshape(B, 2 * OUT_CAP),
      oh0.reshape(B, 2 * HL), oh1.reshape(B, 2 * HL),
      ids_pad, top_p.reshape(1, 1), temperature.reshape(1, 1),
      penalty.reshape(1, 1))


def kernel(hidden_states, input_ids, top_p, temperature, penalty,
           ln_gamma, ln_beta, lm_w):
    ids32 = input_ids.astype(jnp.int32)
    ids_pad = jnp.pad(ids32, ((0, 0), (0, HL - L)))
    lg0 = _logits_chunk(hidden_states, ln_gamma, ln_beta, lm_w, 0, COL0)
    c0 = _sc_collect(lg0, ids_pad, 0, COL0, 0)
    lg1 = _logits_chunk(hidden_states, ln_gamma, ln_beta, lm_w, COL0, W1)
    c1 = _sc_collect(lg1, ids_pad, COL0, W1, 2)
    probs, token = _merge(c0, c1, ids_pad, top_p, temperature, penalty)
    return (probs, token)
